# Initial kernel scaffold; baseline (speedup 1.0000x reference)
import functools

import jax
import jax.numpy as jnp
from jax.experimental import pallas as pl
from jax.experimental.pallas import tpu as pltpu


def _interleave_matrix(tl: int, k: int, dtype):
    rows = k * tl
    i_iota = jax.lax.broadcasted_iota(jnp.int32, (rows, rows), 0)
    j_iota = jax.lax.broadcasted_iota(jnp.int32, (rows, rows), 1)
    o = i_iota // tl
    t = i_iota - o * tl
    return (j_iota == k * t + o).astype(dtype)


def _unpool_kernel(x_ref, idx_ref, o_ref, *, k: int, tl: int):
    x = x_ref[...]
    idx = idx_ref[...]
    tr = x.shape[0]
    t_glob = pl.program_id(1) * tl + jax.lax.broadcasted_iota(
        jnp.int32, (tr, tl), 1)
    base = k * t_glob
    parts = [jnp.where(idx == base + o, x, 0.0) for o in range(k)]
    ed = jnp.concatenate(parts, axis=1)
    g2 = _interleave_matrix(tl, k, x.dtype)
    out = jnp.dot(ed, g2, preferred_element_type=jnp.float32,
                  precision=jax.lax.Precision.HIGH)
    o_ref[...] = out.astype(o_ref.dtype)


def kernel(x, indices):
    k = 2
    N, C, L = x.shape
    Lout = L * k
    rows = N * C
    x2 = x.reshape(rows, L)
    idx2 = indices.reshape(rows, L).astype(jnp.int32)

    TR = 512
    TL = 128
    TN = TL * k
    grid = (rows // TR, L // TL)
    out2 = pl.pallas_call(
        functools.partial(_unpool_kernel, k=k, tl=TL),
        out_shape=jax.ShapeDtypeStruct((rows, Lout), x.dtype),
        grid=grid,
        in_specs=[
            pl.BlockSpec((TR, TL), lambda r, l: (r, l)),
            pl.BlockSpec((TR, TL), lambda r, l: (r, l)),
        ],
        out_specs=pl.BlockSpec((TR, TN), lambda r, l: (r, l)),
        compiler_params=pltpu.CompilerParams(
            dimension_semantics=("parallel", "parallel")),
    )(x2, idx2)
    return out2.reshape(N, C, Lout)


# speedup vs baseline: 1.4371x; 1.4371x over previous
import functools

import jax
import jax.numpy as jnp
from jax.experimental import pallas as pl
from jax.experimental.pallas import tpu as pltpu


def _interleave_matrix(tl: int, k: int, dtype):
    rows = k * tl
    i_iota = jax.lax.broadcasted_iota(jnp.int32, (rows, rows), 0)
    j_iota = jax.lax.broadcasted_iota(jnp.int32, (rows, rows), 1)
    o = i_iota // tl
    t = i_iota - o * tl
    return (j_iota == k * t + o).astype(dtype)


def _unpool_kernel(x_ref, idx_ref, o_ref, *, k: int, tl: int):
    x = x_ref[...]
    idx = idx_ref[...]
    tr = x.shape[0]
    t_glob = pl.program_id(1) * tl + jax.lax.broadcasted_iota(
        jnp.int32, (tr, tl), 1)
    base = k * t_glob
    parts = [jnp.where(idx == base + o, x, 0.0) for o in range(k)]
    ed = jnp.concatenate(parts, axis=1)
    g2 = _interleave_matrix(tl, k, x.dtype)
    out = jnp.dot(ed, g2, preferred_element_type=jnp.float32,
                  precision=jax.lax.Precision.DEFAULT)
    o_ref[...] = out.astype(o_ref.dtype)


def kernel(x, indices):
    k = 2
    N, C, L = x.shape
    Lout = L * k
    rows = N * C
    x2 = x.reshape(rows, L)
    idx2 = indices.reshape(rows, L).astype(jnp.int32)

    TR = 512
    TL = 128
    TN = TL * k
    grid = (rows // TR, L // TL)
    out2 = pl.pallas_call(
        functools.partial(_unpool_kernel, k=k, tl=TL),
        out_shape=jax.ShapeDtypeStruct((rows, Lout), x.dtype),
        grid=grid,
        in_specs=[
            pl.BlockSpec((TR, TL), lambda r, l: (r, l)),
            pl.BlockSpec((TR, TL), lambda r, l: (r, l)),
        ],
        out_specs=pl.BlockSpec((TR, TN), lambda r, l: (r, l)),
        compiler_params=pltpu.CompilerParams(
            dimension_semantics=("parallel", "parallel")),
    )(x2, idx2)
    return out2.reshape(N, C, Lout)


# speedup vs baseline: 2.0454x; 1.4233x over previous
import functools

import jax
import jax.numpy as jnp
from jax.experimental import pallas as pl
from jax.experimental.pallas import tpu as pltpu


def _interleave_matrix(tl: int, k: int, dtype):
    rows = k * tl
    i_iota = jax.lax.broadcasted_iota(jnp.int32, (rows, rows), 0)
    j_iota = jax.lax.broadcasted_iota(jnp.int32, (rows, rows), 1)
    o = i_iota // tl
    t = i_iota - o * tl
    return (j_iota == k * t + o).astype(dtype)


def _unpool_kernel(x_ref, idx_ref, o_ref, *, k: int, tl: int):
    x = x_ref[...]
    idx = idx_ref[...]
    tr = x.shape[0]
    t_glob = pl.program_id(1) * tl + jax.lax.broadcasted_iota(
        jnp.int32, (tr, tl), 1)
    base = k * t_glob
    parts = [jnp.where(idx == base + o, x, 0.0) for o in range(k)]
    ed = jnp.concatenate(parts, axis=1)
    g2 = _interleave_matrix(tl, k, x.dtype)
    out = jnp.dot(ed, g2, preferred_element_type=jnp.float32,
                  precision=jax.lax.Precision.DEFAULT)
    o_ref[...] = out.astype(o_ref.dtype)


def kernel(x, indices):
    k = 2
    N, C, L = x.shape
    Lout = L * k
    rows = N * C
    x2 = x.reshape(rows, L)
    idx2 = indices.reshape(rows, L).astype(jnp.int32)

    TR = 1024
    TL = 128
    TN = TL * k
    grid = (rows // TR, L // TL)
    out2 = pl.pallas_call(
        functools.partial(_unpool_kernel, k=k, tl=TL),
        out_shape=jax.ShapeDtypeStruct((rows, Lout), x.dtype),
        grid=grid,
        in_specs=[
            pl.BlockSpec((TR, TL), lambda r, l: (r, l)),
            pl.BlockSpec((TR, TL), lambda r, l: (r, l)),
        ],
        out_specs=pl.BlockSpec((TR, TN), lambda r, l: (r, l)),
        compiler_params=pltpu.CompilerParams(
            dimension_semantics=("parallel", "parallel")),
    )(x2, idx2)
    return out2.reshape(N, C, Lout)


# speedup vs baseline: 2.9088x; 1.4221x over previous
import functools

import jax
import jax.numpy as jnp
from jax.experimental import pallas as pl
from jax.experimental.pallas import tpu as pltpu


def _interleave_matrix(tl: int, k: int, dtype):
    rows = k * tl
    i_iota = jax.lax.broadcasted_iota(jnp.int32, (rows, rows), 0)
    j_iota = jax.lax.broadcasted_iota(jnp.int32, (rows, rows), 1)
    o = i_iota // tl
    t = i_iota - o * tl
    return (j_iota == k * t + o).astype(dtype)


def _unpool_kernel(x_ref, idx_ref, o_ref, *, k: int, tl: int):
    x = x_ref[...]
    idx = idx_ref[...]
    tr = x.shape[0]
    t_glob = pl.program_id(1) * tl + jax.lax.broadcasted_iota(
        jnp.int32, (tr, tl), 1)
    base = k * t_glob
    parts = [jnp.where(idx == base + o, x, 0.0) for o in range(k)]
    ed = jnp.concatenate(parts, axis=1)
    g2 = _interleave_matrix(tl, k, x.dtype)
    out = jnp.dot(ed, g2, preferred_element_type=jnp.float32,
                  precision=jax.lax.Precision.DEFAULT)
    o_ref[...] = out.astype(o_ref.dtype)


def kernel(x, indices):
    k = 2
    N, C, L = x.shape
    Lout = L * k
    rows = N * C
    x2 = x.reshape(rows, L)
    idx2 = indices.reshape(rows, L).astype(jnp.int32)

    TR = 2048
    TL = 128
    TN = TL * k
    grid = (rows // TR, L // TL)
    out2 = pl.pallas_call(
        functools.partial(_unpool_kernel, k=k, tl=TL),
        out_shape=jax.ShapeDtypeStruct((rows, Lout), x.dtype),
        grid=grid,
        in_specs=[
            pl.BlockSpec((TR, TL), lambda r, l: (r, l)),
            pl.BlockSpec((TR, TL), lambda r, l: (r, l)),
        ],
        out_specs=pl.BlockSpec((TR, TN), lambda r, l: (r, l)),
        compiler_params=pltpu.CompilerParams(
            dimension_semantics=("parallel", "parallel")),
    )(x2, idx2)
    return out2.reshape(N, C, Lout)


# speedup vs baseline: 3.2965x; 1.1333x over previous
import functools

import jax
import jax.numpy as jnp
from jax.experimental import pallas as pl
from jax.experimental.pallas import tpu as pltpu


def _interleave_matrix(tl: int, k: int, dtype):
    rows = k * tl
    i_iota = jax.lax.broadcasted_iota(jnp.int32, (rows, rows), 0)
    j_iota = jax.lax.broadcasted_iota(jnp.int32, (rows, rows), 1)
    o = i_iota // tl
    t = i_iota - o * tl
    return (j_iota == k * t + o).astype(dtype)


def _unpool_kernel(x_ref, idx_ref, o_ref, *, k: int, tl: int):
    x = x_ref[...]
    idx = idx_ref[...]
    tr = x.shape[0]
    t_glob = pl.program_id(1) * tl + jax.lax.broadcasted_iota(
        jnp.int32, (tr, tl), 1)
    base = k * t_glob
    parts = [jnp.where(idx == base + o, x, 0.0) for o in range(k)]
    ed = jnp.concatenate(parts, axis=1)
    g2 = _interleave_matrix(tl, k, x.dtype)
    out = jnp.dot(ed, g2, preferred_element_type=jnp.float32,
                  precision=jax.lax.Precision.DEFAULT)
    o_ref[...] = out.astype(o_ref.dtype)


def kernel(x, indices):
    k = 2
    N, C, L = x.shape
    Lout = L * k
    rows = N * C
    x2 = x.reshape(rows, L)
    idx2 = indices.reshape(rows, L).astype(jnp.int32)

    TR = 4096
    TL = 128
    TN = TL * k
    grid = (rows // TR, L // TL)
    out2 = pl.pallas_call(
        functools.partial(_unpool_kernel, k=k, tl=TL),
        out_shape=jax.ShapeDtypeStruct((rows, Lout), x.dtype),
        grid=grid,
        in_specs=[
            pl.BlockSpec((TR, TL), lambda r, l: (r, l)),
            pl.BlockSpec((TR, TL), lambda r, l: (r, l)),
        ],
        out_specs=pl.BlockSpec((TR, TN), lambda r, l: (r, l)),
        compiler_params=pltpu.CompilerParams(
            dimension_semantics=("parallel", "parallel")),
    )(x2, idx2)
    return out2.reshape(N, C, Lout)


# speedup vs baseline: 3.5215x; 1.0682x over previous
import functools

import jax
import jax.numpy as jnp
from jax.experimental import pallas as pl
from jax.experimental.pallas import tpu as pltpu


def _interleave_matrix(tl: int, k: int, dtype):
    rows = k * tl
    i_iota = jax.lax.broadcasted_iota(jnp.int32, (rows, rows), 0)
    j_iota = jax.lax.broadcasted_iota(jnp.int32, (rows, rows), 1)
    o = i_iota // tl
    t = i_iota - o * tl
    return (j_iota == k * t + o).astype(dtype)


def _unpool_kernel(x_ref, idx_ref, o_ref, *, k: int, tl: int):
    x = x_ref[...]
    idx = idx_ref[...]
    tr = x.shape[0]
    t_glob = pl.program_id(1) * tl + jax.lax.broadcasted_iota(
        jnp.int32, (tr, tl), 1)
    base = k * t_glob
    parts = [jnp.where(idx == base + o, x, 0.0) for o in range(k)]
    ed = jnp.concatenate(parts, axis=1)
    g2 = _interleave_matrix(tl, k, x.dtype)
    out = jnp.dot(ed, g2, preferred_element_type=jnp.float32,
                  precision=jax.lax.Precision.DEFAULT)
    o_ref[...] = out.astype(o_ref.dtype)


def kernel(x, indices):
    k = 2
    N, C, L = x.shape
    Lout = L * k
    rows = N * C
    x2 = x.reshape(rows, L)
    idx2 = indices.reshape(rows, L).astype(jnp.int32)

    TR = 8192
    TL = 128
    TN = TL * k
    grid = (rows // TR, L // TL)
    out2 = pl.pallas_call(
        functools.partial(_unpool_kernel, k=k, tl=TL),
        out_shape=jax.ShapeDtypeStruct((rows, Lout), x.dtype),
        grid=grid,
        in_specs=[
            pl.BlockSpec((TR, TL), lambda r, l: (r, l)),
            pl.BlockSpec((TR, TL), lambda r, l: (r, l)),
        ],
        out_specs=pl.BlockSpec((TR, TN), lambda r, l: (r, l)),
        compiler_params=pltpu.CompilerParams(
            dimension_semantics=("parallel", "parallel"),
            vmem_limit_bytes=100 * 1024 * 1024),
    )(x2, idx2)
    return out2.reshape(N, C, Lout)
